# baseline (speedup 1.0000x reference)
import functools

import jax
import jax.numpy as jnp
from jax.experimental import pallas as pl
from jax.experimental.pallas import tpu as pltpu


def _loss_table_kernel(emb_ref, embl_ref):
    x = emb_ref[...]
    m = jnp.max(x, axis=-1, keepdims=True)
    s = jnp.sum(jnp.exp(x - m), axis=-1, keepdims=True)
    lse = m + jnp.log(s)
    embl_ref[...] = lse - x


def _gather_kernel(toks_ref, q_ref, l_ref, emb_ref, lossF_ref,
                   out_hbm, loss_hbm, s_ref, obuf_ref, lacc_ref,
                   sems, loss_sem, *, rows, slots):
    C = emb_ref.shape[2] // 128
    S = rows + 1
    G = rows // 8
    i = pl.program_id(0)
    nb = pl.num_programs(0)
    slot = jax.lax.rem(i, slots)
    lane = jax.lax.broadcasted_iota(jnp.int32, (1, 128), 1)

    @pl.when(i >= slots)
    def _():
        pltpu.make_async_copy(
            obuf_ref.at[slot],
            out_hbm.at[pl.ds((i - slots) * G, G)],
            sems.at[slot]).wait()

    acc = [jnp.zeros((1, 128), jnp.float32) for _ in range(2)]
    for r in range(rows):
        tok = toks_ref[0, 0, r]
        q = q_ref[0, 0, r]
        lidx = l_ref[0, 0, r]
        row = emb_ref[tok, 0].reshape(C, 128)
        s_ref[r:r + S * C:S, :] = row
        tile = lossF_ref[q, 0:1, :]
        acc[r & 1] = acc[r & 1] + jnp.where(lane == lidx, tile, 0.0)
    total = acc[0] + acc[1]
    lacc_ref[...] = jnp.where(i == 0, total, lacc_ref[...] + total)
    for c in range(C):
        val = s_ref[pl.ds(S * c, rows), :]
        obuf_ref[slot, :, :, pl.ds(c * 128, 128)] = val.reshape(G, 8, 128)
    pltpu.make_async_copy(
        obuf_ref.at[slot], out_hbm.at[pl.ds(i * G, G)], sems.at[slot]).start()

    @pl.when(i == nb - 1)
    def _():
        pltpu.make_async_copy(lacc_ref, loss_hbm, loss_sem).start()
        pltpu.make_async_copy(lacc_ref, loss_hbm, loss_sem).wait()
        for s in range(min(slots, 1024)):
            @pl.when(jnp.logical_and(s != slot, i >= s))
            def _(s=s):
                pltpu.make_async_copy(
                    obuf_ref.at[s], out_hbm.at[pl.ds(0, G)],
                    sems.at[s]).wait()
        pltpu.make_async_copy(
            obuf_ref.at[slot], out_hbm.at[pl.ds(i * G, G)],
            sems.at[slot]).wait()


def kernel(toks, next_toks, emb):
    V = emb.shape[0]
    B, T = toks.shape
    N = B * T

    toks_flat = jnp.clip(toks.reshape(N).astype(jnp.int32), 0, V - 1)
    next_flat = jnp.clip(next_toks.reshape(N).astype(jnp.int32), 0, V - 1)

    rb = 128 if V % 128 == 0 else 8
    embl = pl.pallas_call(
        _loss_table_kernel,
        grid=(V // rb,),
        in_specs=[pl.BlockSpec((rb, V), lambda i: (i, 0))],
        out_specs=pl.BlockSpec((rb, V), lambda i: (i, 0)),
        out_shape=jax.ShapeDtypeStruct((V, V), jnp.float32),
        compiler_params=pltpu.CompilerParams(
            dimension_semantics=("parallel",)),
    )(emb)

    R = 256
    while N % R:
        R //= 2
    nb = N // R

    emb3 = emb.reshape(V, 1, V)
    lossF = embl.reshape(V * V // 128, 1, 128)
    pair = toks_flat * V + next_flat
    toks2 = toks_flat.reshape(nb, 1, R)
    q2 = (pair >> 7).reshape(nb, 1, R)
    l2 = (pair & 127).reshape(nb, 1, R)

    SLOTS = 6
    logits3, loss_parts = pl.pallas_call(
        functools.partial(_gather_kernel, rows=R, slots=SLOTS),
        grid=(nb,),
        in_specs=[
            pl.BlockSpec((1, 1, R), lambda i: (i, 0, 0),
                         memory_space=pltpu.SMEM),
            pl.BlockSpec((1, 1, R), lambda i: (i, 0, 0),
                         memory_space=pltpu.SMEM),
            pl.BlockSpec((1, 1, R), lambda i: (i, 0, 0),
                         memory_space=pltpu.SMEM),
            pl.BlockSpec(memory_space=pltpu.VMEM),
            pl.BlockSpec(memory_space=pltpu.VMEM),
        ],
        out_specs=(
            pl.BlockSpec(memory_space=pl.ANY),
            pl.BlockSpec(memory_space=pl.ANY),
        ),
        out_shape=(
            jax.ShapeDtypeStruct((N // 8, 8, V), jnp.float32),
            jax.ShapeDtypeStruct((1, 128), jnp.float32),
        ),
        scratch_shapes=[
            pltpu.VMEM(((R + 1) * (V // 128), 128), jnp.float32),
            pltpu.VMEM((SLOTS, R // 8, 8, V), jnp.float32),
            pltpu.VMEM((1, 128), jnp.float32),
            pltpu.SemaphoreType.DMA((SLOTS,)),
            pltpu.SemaphoreType.DMA,
        ],
        compiler_params=pltpu.CompilerParams(
            dimension_semantics=("arbitrary",),
            vmem_limit_bytes=64 * 1024 * 1024),
        cost_estimate=pl.CostEstimate(
            flops=2 * N * V,
            transcendentals=0,
            bytes_accessed=(2 * V * V + 2 * N * V) * 4),
    )(toks2, q2, l2, emb3, lossF)

    logits = logits3.reshape(B, T, V)
    loss = jnp.sum(loss_parts) / N
    return logits, loss


# speedup vs baseline: 1.0048x; 1.0048x over previous
import functools

import jax
import jax.numpy as jnp
from jax.experimental import pallas as pl
from jax.experimental.pallas import tpu as pltpu


def _loss_table_kernel(emb_ref, embl_ref):
    x = emb_ref[...]
    m = jnp.max(x, axis=-1, keepdims=True)
    s = jnp.sum(jnp.exp(x - m), axis=-1, keepdims=True)
    lse = m + jnp.log(s)
    embl_ref[...] = lse - x


def _gather_kernel(toks_ref, q_ref, l_ref, emb_ref, lossF_ref,
                   out_hbm, loss_hbm, s_ref, obuf_ref, lacc_ref,
                   sems, loss_sem, *, rows, slots):
    C = emb_ref.shape[2] // 128
    S = rows + 1
    G = rows // 8
    i = pl.program_id(0)
    nb = pl.num_programs(0)
    slot = jax.lax.rem(i, slots)
    lane = jax.lax.broadcasted_iota(jnp.int32, (1, 128), 1)

    @pl.when(i >= slots)
    def _():
        pltpu.make_async_copy(
            obuf_ref.at[slot],
            out_hbm.at[pl.ds((i - slots) * G, G)],
            sems.at[slot]).wait()

    acc = [jnp.zeros((1, 128), jnp.float32) for _ in range(2)]
    for r in range(rows):
        tok = toks_ref[0, 0, r]
        q = q_ref[0, 0, r]
        lidx = l_ref[0, 0, r]
        row = emb_ref[tok, 0].reshape(C, 128)
        s_ref[r:r + S * C:S, :] = row
        tile = lossF_ref[q, 0:1, :]
        acc[r & 1] = acc[r & 1] + jnp.where(lane == lidx, tile, 0.0)
    total = acc[0] + acc[1]
    lacc_ref[...] = jnp.where(i == 0, total, lacc_ref[...] + total)
    for c in range(C):
        val = s_ref[pl.ds(S * c, rows), :]
        obuf_ref[slot, :, :, pl.ds(c * 128, 128)] = val.reshape(G, 8, 128)
    pltpu.make_async_copy(
        obuf_ref.at[slot], out_hbm.at[pl.ds(i * G, G)], sems.at[slot]).start()

    @pl.when(i == nb - 1)
    def _():
        pltpu.make_async_copy(lacc_ref, loss_hbm, loss_sem).start()
        pltpu.make_async_copy(lacc_ref, loss_hbm, loss_sem).wait()
        for s in range(min(slots, 1024)):
            @pl.when(jnp.logical_and(s != slot, i >= s))
            def _(s=s):
                pltpu.make_async_copy(
                    obuf_ref.at[s], out_hbm.at[pl.ds(0, G)],
                    sems.at[s]).wait()
        pltpu.make_async_copy(
            obuf_ref.at[slot], out_hbm.at[pl.ds(i * G, G)],
            sems.at[slot]).wait()


def kernel(toks, next_toks, emb):
    V = emb.shape[0]
    B, T = toks.shape
    N = B * T

    toks_flat = jnp.clip(toks.reshape(N).astype(jnp.int32), 0, V - 1)
    next_flat = jnp.clip(next_toks.reshape(N).astype(jnp.int32), 0, V - 1)

    rb = 128 if V % 128 == 0 else 8
    embl = pl.pallas_call(
        _loss_table_kernel,
        grid=(V // rb,),
        in_specs=[pl.BlockSpec((rb, V), lambda i: (i, 0))],
        out_specs=pl.BlockSpec((rb, V), lambda i: (i, 0)),
        out_shape=jax.ShapeDtypeStruct((V, V), jnp.float32),
        compiler_params=pltpu.CompilerParams(
            dimension_semantics=("parallel",)),
    )(emb)

    R = 256
    while N % R:
        R //= 2
    nb = N // R

    emb3 = emb.reshape(V, 1, V)
    lossF = embl.reshape(V * V // 128, 1, 128)
    pair = toks_flat * V + next_flat
    toks2 = toks_flat.reshape(nb, 1, R)
    q2 = (pair >> 7).reshape(nb, 1, R)
    l2 = (pair & 127).reshape(nb, 1, R)

    SLOTS = 4
    logits3, loss_parts = pl.pallas_call(
        functools.partial(_gather_kernel, rows=R, slots=SLOTS),
        grid=(nb,),
        in_specs=[
            pl.BlockSpec((1, 1, R), lambda i: (i, 0, 0),
                         memory_space=pltpu.SMEM),
            pl.BlockSpec((1, 1, R), lambda i: (i, 0, 0),
                         memory_space=pltpu.SMEM),
            pl.BlockSpec((1, 1, R), lambda i: (i, 0, 0),
                         memory_space=pltpu.SMEM),
            pl.BlockSpec(memory_space=pltpu.VMEM),
            pl.BlockSpec(memory_space=pltpu.VMEM),
        ],
        out_specs=(
            pl.BlockSpec(memory_space=pl.ANY),
            pl.BlockSpec(memory_space=pl.ANY),
        ),
        out_shape=(
            jax.ShapeDtypeStruct((N // 8, 8, V), jnp.float32),
            jax.ShapeDtypeStruct((1, 128), jnp.float32),
        ),
        scratch_shapes=[
            pltpu.VMEM(((R + 1) * (V // 128), 128), jnp.float32),
            pltpu.VMEM((SLOTS, R // 8, 8, V), jnp.float32),
            pltpu.VMEM((1, 128), jnp.float32),
            pltpu.SemaphoreType.DMA((SLOTS,)),
            pltpu.SemaphoreType.DMA,
        ],
        compiler_params=pltpu.CompilerParams(
            dimension_semantics=("arbitrary",),
            vmem_limit_bytes=64 * 1024 * 1024),
        cost_estimate=pl.CostEstimate(
            flops=2 * N * V,
            transcendentals=0,
            bytes_accessed=(2 * V * V + 2 * N * V) * 4),
    )(toks2, q2, l2, emb3, lossF)

    logits = logits3.reshape(B, T, V)
    loss = jnp.sum(loss_parts) / N
    return logits, loss
